# Initial kernel scaffold; baseline (speedup 1.0000x reference)
import jax
import jax.numpy as jnp
from jax.experimental import pallas as pl


def kernel(x, p2e):
    raise NotImplementedError("write your pallas kernel here")


# speedup vs baseline: 2.1449x; 2.1449x over previous
import functools

import jax
import jax.numpy as jnp
from jax import lax
from jax.experimental import pallas as pl
from jax.experimental.pallas import tpu as pltpu
from jax.experimental.pallas import tpu_sc as plsc

D_MODEL = 1024
NUM_CORES = 2
NUM_SUBCORES = 16
NUM_WORKERS = NUM_CORES * NUM_SUBCORES


@functools.lru_cache(maxsize=None)
def _make_gather(B: int, C: int):
    b_per_w = B // NUM_WORKERS
    n_chunks = b_per_w // C
    mesh = plsc.VectorSubcoreMesh(
        core_axis_name="c",
        subcore_axis_name="s",
        num_cores=NUM_CORES,
        num_subcores=NUM_SUBCORES,
    )

    @functools.partial(
        pl.kernel,
        out_type=jax.ShapeDtypeStruct((B, D_MODEL), jnp.float32),
        mesh=mesh,
        scratch_types=[
            pltpu.VMEM((b_per_w,), jnp.int32),
            pltpu.VMEM((C, D_MODEL), jnp.float32),
            pltpu.SemaphoreType.DMA,
        ],
    )
    def gather_kernel(table_hbm, idx_hbm, out_hbm, idx_v, rows_v, sem):
        wid = lax.axis_index("s") * NUM_CORES + lax.axis_index("c")
        base = wid * b_per_w
        pltpu.sync_copy(idx_hbm.at[pl.ds(base, b_per_w)], idx_v)
        for j in range(n_chunks):
            pltpu.async_copy(
                table_hbm.at[idx_v.at[pl.ds(j * C, C)]], rows_v, sem
            ).wait()
            pltpu.sync_copy(rows_v, out_hbm.at[pl.ds(base + j * C, C)])

    return gather_kernel


def kernel(x, p2e):
    shp = x.shape
    idx = x.reshape(-1).astype(jnp.int32)
    out = _make_gather(idx.shape[0], 64)(p2e, idx)
    return out.reshape(shp + (D_MODEL,))


# speedup vs baseline: 2.3162x; 1.0799x over previous
import functools

import jax
import jax.numpy as jnp
from jax import lax
from jax.experimental import pallas as pl
from jax.experimental.pallas import tpu as pltpu
from jax.experimental.pallas import tpu_sc as plsc

D_MODEL = 1024
NUM_CORES = 2
NUM_SUBCORES = 16
NUM_WORKERS = NUM_CORES * NUM_SUBCORES


@functools.lru_cache(maxsize=None)
def _make_gather(B: int, C: int):
    b_per_w = B // NUM_WORKERS
    n_chunks = b_per_w // C
    mesh = plsc.VectorSubcoreMesh(
        core_axis_name="c",
        subcore_axis_name="s",
        num_cores=NUM_CORES,
        num_subcores=NUM_SUBCORES,
    )

    @functools.partial(
        pl.kernel,
        out_type=jax.ShapeDtypeStruct((B, D_MODEL), jnp.float32),
        mesh=mesh,
        scratch_types=[
            pltpu.VMEM((b_per_w,), jnp.int32),
            pltpu.VMEM((C, D_MODEL), jnp.float32),
            pltpu.VMEM((C, D_MODEL), jnp.float32),
            pltpu.SemaphoreType.DMA,
            pltpu.SemaphoreType.DMA,
            pltpu.SemaphoreType.DMA,
            pltpu.SemaphoreType.DMA,
        ],
    )
    def gather_kernel(table_hbm, idx_hbm, out_hbm, idx_v, rows0, rows1,
                      g0, g1, w0, w1):
        wid = lax.axis_index("s") * NUM_CORES + lax.axis_index("c")
        base = wid * b_per_w
        pltpu.sync_copy(idx_hbm.at[pl.ds(base, b_per_w)], idx_v)
        bufs, gsems, wsems = (rows0, rows1), (g0, g1), (w0, w1)

        def start_gather(c):
            b = c % 2
            return pltpu.async_copy(
                table_hbm.at[idx_v.at[pl.ds(c * C, C)]], bufs[b], gsems[b]
            )

        gops = [None] * n_chunks
        wops = [None] * n_chunks
        gops[0] = start_gather(0)
        gops[1] = start_gather(1)
        for c in range(n_chunks):
            b = c % 2
            gops[c].wait()
            wops[c] = pltpu.async_copy(
                bufs[b], out_hbm.at[pl.ds(base + c * C, C)], wsems[b]
            )
            if c + 2 < n_chunks:
                wops[c].wait()
                gops[c + 2] = start_gather(c + 2)
        wops[n_chunks - 2].wait()
        wops[n_chunks - 1].wait()

    return gather_kernel


def kernel(x, p2e):
    shp = x.shape
    idx = x.reshape(-1).astype(jnp.int32)
    out = _make_gather(idx.shape[0], 32)(p2e, idx)
    return out.reshape(shp + (D_MODEL,))


# speedup vs baseline: 2.3252x; 1.0039x over previous
import functools

import jax
import jax.numpy as jnp
from jax import lax
from jax.experimental import pallas as pl
from jax.experimental.pallas import tpu as pltpu
from jax.experimental.pallas import tpu_sc as plsc

D_MODEL = 1024
NUM_CORES = 2
NUM_SUBCORES = 16
NUM_WORKERS = NUM_CORES * NUM_SUBCORES


@functools.lru_cache(maxsize=None)
def _make_gather(B: int, C: int, NBUF: int):
    b_per_w = B // NUM_WORKERS
    n_chunks = b_per_w // C
    mesh = plsc.VectorSubcoreMesh(
        core_axis_name="c",
        subcore_axis_name="s",
        num_cores=NUM_CORES,
        num_subcores=NUM_SUBCORES,
    )

    @functools.partial(
        pl.kernel,
        out_type=jax.ShapeDtypeStruct((B, D_MODEL), jnp.float32),
        mesh=mesh,
        scratch_types=[
            pltpu.VMEM((b_per_w,), jnp.int32),
            pltpu.VMEM((NBUF, C, D_MODEL), jnp.float32),
            [pltpu.SemaphoreType.DMA] * NBUF,
            [pltpu.SemaphoreType.DMA] * NBUF,
        ],
    )
    def gather_kernel(table_hbm, idx_hbm, out_hbm, idx_v, rows, gsems, wsems):
        wid = lax.axis_index("s") * NUM_CORES + lax.axis_index("c")
        base = wid * b_per_w
        pltpu.sync_copy(idx_hbm.at[pl.ds(base, b_per_w)], idx_v)

        def start_gather(c):
            b = c % NBUF
            return pltpu.async_copy(
                table_hbm.at[idx_v.at[pl.ds(c * C, C)]], rows.at[b], gsems[b]
            )

        gops = [None] * n_chunks
        wops = [None] * n_chunks
        for c in range(min(NBUF, n_chunks)):
            gops[c] = start_gather(c)
        for c in range(n_chunks):
            b = c % NBUF
            gops[c].wait()
            wops[c] = pltpu.async_copy(
                rows.at[b], out_hbm.at[pl.ds(base + c * C, C)], wsems[b]
            )
            if c + NBUF < n_chunks:
                wops[c].wait()
                gops[c + NBUF] = start_gather(c + NBUF)
        for c in range(max(0, n_chunks - NBUF), n_chunks):
            wops[c].wait()

    return gather_kernel


def kernel(x, p2e):
    shp = x.shape
    idx = x.reshape(-1).astype(jnp.int32)
    out = _make_gather(idx.shape[0], 32, 3)(p2e, idx)
    return out.reshape(shp + (D_MODEL,))
